# baseline (speedup 1.0000x reference)
import functools

import jax
import jax.numpy as jnp
from jax import lax
from jax.experimental import pallas as pl
from jax.experimental.pallas import tpu as pltpu
from jax.experimental.pallas import tpu_sc as plsc

B = 16384
D = 64
L = 16


def _make_sc_kernel(num_cores, num_subcores):
    NW = num_cores * num_subcores
    bw = B // NW
    mesh = plsc.VectorSubcoreMesh(core_axis_name="c", subcore_axis_name="s")

    @functools.partial(
        pl.kernel,
        mesh=mesh,
        out_type=jax.ShapeDtypeStruct((B,), jnp.float32),
        scratch_types=[
            pltpu.VMEM((bw,), jnp.int32),
            pltpu.VMEM((bw,), jnp.int32),
            pltpu.VMEM((bw, D), jnp.float32),
            pltpu.VMEM((bw, D), jnp.float32),
            pltpu.VMEM((bw,), jnp.float32),
            pltpu.VMEM((L * L,), jnp.float32),
            pltpu.SemaphoreType.DMA,
            pltpu.SemaphoreType.DMA,
        ],
        compiler_params=pltpu.CompilerParams(
            needs_layout_passes=False, use_tc_tiling_on_sc=False),
    )
    def k(uidx_hbm, midx_hbm, ut_hbm, mt_hbm, out_hbm,
          uidx_v, midx_v, urows_v, mrows_v, out_v, accbuf_v, sem_u, sem_m):
        wid = lax.axis_index("s") * num_cores + lax.axis_index("c")
        base = wid * bw
        pltpu.sync_copy(uidx_hbm.at[pl.ds(base, bw)], uidx_v)
        pltpu.sync_copy(midx_hbm.at[pl.ds(base, bw)], midx_v)
        cu = pltpu.async_copy(ut_hbm.at[uidx_v], urows_v, sem_u)
        cm = pltpu.async_copy(mt_hbm.at[midx_v], mrows_v, sem_m)
        cu.wait()
        cm.wait()

        riota = lax.iota(jnp.int32, L)

        def body(g, carry):
            for j in range(L):
                r = g * L + j
                acc = urows_v[r, pl.ds(0, L)] * mrows_v[r, pl.ds(0, L)]
                for k in range(1, D // L):
                    acc = acc + (urows_v[r, pl.ds(k * L, L)]
                                 * mrows_v[r, pl.ds(k * L, L)])
                accbuf_v[pl.ds(j * L, L)] = acc
            res = jnp.zeros((L,), jnp.float32)
            for i in range(L):
                res = res + plsc.load_gather(accbuf_v, [riota * L + i])
            out_v[pl.ds(g * L, L)] = res
            return carry

        lax.fori_loop(0, bw // L, body, 0)
        pltpu.sync_copy(out_v, out_hbm.at[pl.ds(base, bw)])

    return k


def kernel(inputs, user_table, movie_table):
    info = plsc.get_sparse_core_info()
    k = _make_sc_kernel(info.num_cores, info.num_subcores)
    user_idx = inputs[:, 0]
    movie_idx = inputs[:, 1]
    out = k(user_idx, movie_idx, user_table, movie_table)
    return out.reshape(B, 1)


# speedup vs baseline: 1.5625x; 1.5625x over previous
import functools

import jax
import jax.numpy as jnp
from jax import lax
from jax.experimental import pallas as pl
from jax.experimental.pallas import tpu as pltpu
from jax.experimental.pallas import tpu_sc as plsc

B = 16384
D = 64
L = 16
CH = 256


def _make_sc_kernel(num_cores, num_subcores):
    NW = num_cores * num_subcores
    bw = B // NW
    mesh = plsc.VectorSubcoreMesh(core_axis_name="c", subcore_axis_name="s")

    @functools.partial(
        pl.kernel,
        mesh=mesh,
        out_type=jax.ShapeDtypeStruct((B,), jnp.float32),
        scratch_types=[
            pltpu.VMEM((bw,), jnp.int32),
            pltpu.VMEM((bw,), jnp.int32),
            pltpu.VMEM((CH, D), jnp.float32),
            pltpu.VMEM((CH, D), jnp.float32),
            pltpu.VMEM((bw,), jnp.float32),
            pltpu.VMEM((L * L,), jnp.float32),
            pltpu.SemaphoreType.DMA,
            pltpu.SemaphoreType.DMA,
        ],
        compiler_params=pltpu.CompilerParams(needs_layout_passes=False),
    )
    def k(uidx_hbm, midx_hbm, ut_hbm, mt_hbm, out_hbm,
          uidx_v, midx_v, urows_v, mrows_v, out_v, accbuf_v,
          sem_u, sem_m):
        wid = lax.axis_index("s") * num_cores + lax.axis_index("c")
        base = wid * bw
        pltpu.sync_copy(uidx_hbm.at[pl.ds(base, bw)], uidx_v)
        pltpu.sync_copy(midx_hbm.at[pl.ds(base, bw)], midx_v)

        riota = lax.iota(jnp.int32, L)

        def chunk(c, carry):
            c0 = c * CH

            def issue(g, carry2):
                ivu = uidx_v[pl.ds(c0 + g * L, L)]
                ivm = midx_v[pl.ds(c0 + g * L, L)]
                for j in range(L):
                    ru = ivu[j]
                    rm = ivm[j]
                    pltpu.make_async_copy(
                        ut_hbm.at[pl.ds(ru, 1)],
                        urows_v.at[pl.ds(g * L + j, 1)], sem_u).start()
                    pltpu.make_async_copy(
                        mt_hbm.at[pl.ds(rm, 1)],
                        mrows_v.at[pl.ds(g * L + j, 1)], sem_m).start()
                return carry2

            lax.fori_loop(0, CH // L, issue, 0)

            def drain(j, carry2):
                pltpu.make_async_copy(
                    ut_hbm.at[pl.ds(0, 1)], urows_v.at[pl.ds(0, 1)],
                    sem_u).wait()
                pltpu.make_async_copy(
                    mt_hbm.at[pl.ds(0, 1)], mrows_v.at[pl.ds(0, 1)],
                    sem_m).wait()
                return carry2

            lax.fori_loop(0, CH, drain, 0)

            def body(g, carry2):
                for j in range(L):
                    r = g * L + j
                    acc = urows_v[r, pl.ds(0, L)] * mrows_v[r, pl.ds(0, L)]
                    for kk in range(1, D // L):
                        acc = acc + (urows_v[r, pl.ds(kk * L, L)]
                                     * mrows_v[r, pl.ds(kk * L, L)])
                    accbuf_v[pl.ds(j * L, L)] = acc
                res = jnp.zeros((L,), jnp.float32)
                for i in range(L):
                    res = res + plsc.load_gather(accbuf_v, [riota * L + i])
                out_v[pl.ds(c0 + g * L, L)] = res
                return carry2

            lax.fori_loop(0, CH // L, body, 0)
            return carry

        lax.fori_loop(0, bw // CH, chunk, 0)
        pltpu.sync_copy(out_v, out_hbm.at[pl.ds(base, bw)])

    return k


def kernel(inputs, user_table, movie_table):
    info = plsc.get_sparse_core_info()
    k = _make_sc_kernel(info.num_cores, info.num_subcores)
    user_idx = inputs[:, 0]
    movie_idx = inputs[:, 1]
    out = k(user_idx, movie_idx, user_table, movie_table)
    return out.reshape(B, 1)
